# baseline (speedup 1.0000x reference)
import jax
import jax.numpy as jnp
from jax.experimental import pallas as pl
from jax.experimental.pallas import tpu as pltpu

_DIM = 1024
_HID = 4096
_E = 8
_K = 2
_N = 2048
_BT = 256
_TMAX = (_N * _K) // _BT + _E
_P = _TMAX * _BT
_HB = 2048
_NHB = _HID // _HB


def _ffn_body(be_ref, tok_ref, wp_ref, nlive_ref,
              x_ref, w1_ref, w2_ref, b1_ref, b2_ref,
              out_ref, xg_ref, acc_ref):
    h = pl.program_id(0)
    t = pl.program_id(1)

    @pl.when((h == 0) & (t == 0))
    def _init():
        out_ref[...] = jnp.zeros_like(out_ref)

    @pl.when(t < nlive_ref[0])
    def _work():
        base = t * _BT

        def gather(i, carry):
            tok = tok_ref[base + i]
            xg_ref[pl.ds(i, 1), :] = x_ref[pl.ds(tok, 1), :]
            return carry

        jax.lax.fori_loop(0, _BT, gather, 0)

        hid = jax.lax.dot_general(
            xg_ref[...], w1_ref[0], (((1,), (1,)), ((), ())),
            preferred_element_type=jnp.float32)
        hid = jnp.maximum(hid + b1_ref[0], 0.0)
        oe = jax.lax.dot_general(
            hid, w2_ref[0], (((1,), (1,)), ((), ())),
            preferred_element_type=jnp.float32)
        acc_ref[...] = jnp.where(h == 0, oe + b2_ref[0], oe)

        def scatter(i, carry):
            tok = tok_ref[base + i]
            w = wp_ref[base + i]
            out_ref[pl.ds(tok, 1), :] += w * acc_ref[pl.ds(i, 1), :]
            return carry

        jax.lax.fori_loop(0, _BT, scatter, 0)


def kernel(x, gate_w, noise_w, W1, b1, W2, b2):
    eps = jax.random.normal(jax.random.key(42), (_N, _E), dtype=jnp.float32)

    g = x @ gate_w.T
    nstd = jax.nn.softplus(x @ noise_w.T)
    h = g + eps * nstd
    _, low_idx = jax.lax.top_k(-h, _K)
    rows = jnp.arange(_N)[:, None]
    h = h.at[rows, low_idx].set(-jnp.inf)
    L = jax.nn.softmax(h, axis=1)
    wts, idx = jax.lax.top_k(L, _K)
    tmp = L.sum(axis=0)
    loss = (jnp.std(tmp, ddof=1) / jnp.mean(tmp)) ** 2

    ek = idx.reshape(-1)
    order = jnp.argsort(ek, stable=True).astype(jnp.int32)
    counts = jnp.bincount(ek, length=_E)
    offs = jnp.cumsum(counts) - counts
    ptiles = (counts + _BT - 1) // _BT
    pt_end = jnp.cumsum(ptiles)
    pt_off = (pt_end - ptiles) * _BT
    e_sorted = ek[order]
    ranks = jnp.arange(_N * _K, dtype=jnp.int32) - offs[e_sorted]
    pos = pt_off[e_sorted] + ranks
    tokpad = jnp.zeros((_P,), jnp.int32).at[pos].set(order // _K)
    wpad = jnp.zeros((_P,), jnp.float32).at[pos].set(wts.reshape(-1)[order])
    n_live = pt_end[-1].astype(jnp.int32)
    tile_ids = jnp.arange(_TMAX)
    be = jnp.searchsorted(pt_end, tile_ids, side="right").astype(jnp.int32)
    be_last = be[jnp.maximum(n_live - 1, 0)]
    be = jnp.where(tile_ids < n_live, jnp.minimum(be, _E - 1), be_last)

    grid_spec = pltpu.PrefetchScalarGridSpec(
        num_scalar_prefetch=4,
        grid=(_NHB, _TMAX),
        in_specs=[
            pl.BlockSpec((_N, _DIM),
                         lambda h, t, be_r, tok_r, wp_r, nl_r: (0, 0)),
            pl.BlockSpec((1, _HB, _DIM),
                         lambda h, t, be_r, tok_r, wp_r, nl_r: (be_r[t], h, 0)),
            pl.BlockSpec((1, _DIM, _HB),
                         lambda h, t, be_r, tok_r, wp_r, nl_r: (be_r[t], 0, h)),
            pl.BlockSpec((1, 1, _HB),
                         lambda h, t, be_r, tok_r, wp_r, nl_r: (be_r[t], 0, h)),
            pl.BlockSpec((1, 1, _DIM),
                         lambda h, t, be_r, tok_r, wp_r, nl_r: (be_r[t], 0, 0)),
        ],
        out_specs=pl.BlockSpec((_N, _DIM),
                               lambda h, t, be_r, tok_r, wp_r, nl_r: (0, 0)),
        scratch_shapes=[
            pltpu.VMEM((_BT, _DIM), jnp.float32),
            pltpu.VMEM((_BT, _DIM), jnp.float32),
        ],
    )
    out = pl.pallas_call(
        _ffn_body,
        grid_spec=grid_spec,
        out_shape=jax.ShapeDtypeStruct((_N, _DIM), jnp.float32),
    )(be, tokpad, wpad, n_live[None], x, W1, W2,
      b1.reshape(_E, 1, _HID), b2.reshape(_E, 1, _DIM))

    return out, loss


# speedup vs baseline: 1.0019x; 1.0019x over previous
import jax
import jax.numpy as jnp
from jax.experimental import pallas as pl
from jax.experimental.pallas import tpu as pltpu

_DIM = 1024
_HID = 4096
_E = 8
_K = 2
_N = 2048
_BT = 256
_TMAX = (_N * _K) // _BT + _E
_P = _TMAX * _BT
_HB = 2048
_NHB = _HID // _HB


def _ffn_body(be_ref, tok_ref, wp_ref, nlive_ref,
              x_ref, w1_ref, w2_ref, b1_ref, b2_ref,
              out_ref, xg0_ref, xg1_ref, acc_ref):
    h = pl.program_id(0)
    t = pl.program_id(1)
    s = h * _TMAX + t

    def gather_tile(tile, xg):
        base = tile * _BT

        def body(i, carry):
            tok = tok_ref[base + i]
            xg[pl.ds(i, 1), :] = x_ref[pl.ds(tok, 1), :]
            return carry

        jax.lax.fori_loop(0, _BT, body, 0)

    @pl.when(s == 0)
    def _init():
        out_ref[...] = jnp.zeros_like(out_ref)
        gather_tile(0, xg0_ref)

    def ffn(xg):
        hid = jax.lax.dot_general(
            xg[...], w1_ref[0], (((1,), (1,)), ((), ())),
            preferred_element_type=jnp.float32)
        hid = jnp.maximum(hid + b1_ref[0], 0.0)
        oe = jax.lax.dot_general(
            hid, w2_ref[0], (((1,), (1,)), ((), ())),
            preferred_element_type=jnp.float32)
        acc_ref[...] = jnp.where(h == 0, oe + b2_ref[0], oe)

    @pl.when((t < nlive_ref[0]) & (t % 2 == 0))
    def _ffn_even():
        ffn(xg0_ref)

    @pl.when((t < nlive_ref[0]) & (t % 2 == 1))
    def _ffn_odd():
        ffn(xg1_ref)

    t_next = jnp.where(t == _TMAX - 1, 0, t + 1)
    nxt = (t_next < nlive_ref[0]) & (s < _NHB * _TMAX - 1)

    @pl.when(nxt & (t % 2 == 1))
    def _stage_even():
        gather_tile(t_next, xg0_ref)

    @pl.when(nxt & (t % 2 == 0))
    def _stage_odd():
        gather_tile(t_next, xg1_ref)

    @pl.when(t < nlive_ref[0])
    def _scatter():
        base = t * _BT

        def body(i, carry):
            tok = tok_ref[base + i]
            w = wp_ref[base + i]
            out_ref[pl.ds(tok, 1), :] += w * acc_ref[pl.ds(i, 1), :]
            return carry

        jax.lax.fori_loop(0, _BT, body, 0)


def kernel(x, gate_w, noise_w, W1, b1, W2, b2):
    eps = jax.random.normal(jax.random.key(42), (_N, _E), dtype=jnp.float32)

    g = x @ gate_w.T
    nstd = jax.nn.softplus(x @ noise_w.T)
    h = g + eps * nstd
    _, low_idx = jax.lax.top_k(-h, _K)
    rows = jnp.arange(_N)[:, None]
    h = h.at[rows, low_idx].set(-jnp.inf)
    L = jax.nn.softmax(h, axis=1)
    wts, idx = jax.lax.top_k(L, _K)
    tmp = L.sum(axis=0)
    loss = (jnp.std(tmp, ddof=1) / jnp.mean(tmp)) ** 2

    ek = idx.reshape(-1)
    order = jnp.argsort(ek, stable=True).astype(jnp.int32)
    counts = jnp.bincount(ek, length=_E)
    offs = jnp.cumsum(counts) - counts
    ptiles = (counts + _BT - 1) // _BT
    pt_end = jnp.cumsum(ptiles)
    pt_off = (pt_end - ptiles) * _BT
    e_sorted = ek[order]
    ranks = jnp.arange(_N * _K, dtype=jnp.int32) - offs[e_sorted]
    pos = pt_off[e_sorted] + ranks
    tokpad = jnp.zeros((_P,), jnp.int32).at[pos].set(order // _K)
    wpad = jnp.zeros((_P,), jnp.float32).at[pos].set(wts.reshape(-1)[order])
    n_live = pt_end[-1].astype(jnp.int32)
    tile_ids = jnp.arange(_TMAX)
    be = jnp.searchsorted(pt_end, tile_ids, side="right").astype(jnp.int32)
    be_last = be[jnp.maximum(n_live - 1, 0)]
    be = jnp.where(tile_ids < n_live, jnp.minimum(be, _E - 1), be_last)

    grid_spec = pltpu.PrefetchScalarGridSpec(
        num_scalar_prefetch=4,
        grid=(_NHB, _TMAX),
        in_specs=[
            pl.BlockSpec((_N, _DIM),
                         lambda h, t, be_r, tok_r, wp_r, nl_r: (0, 0)),
            pl.BlockSpec((1, _HB, _DIM),
                         lambda h, t, be_r, tok_r, wp_r, nl_r: (be_r[t], h, 0)),
            pl.BlockSpec((1, _DIM, _HB),
                         lambda h, t, be_r, tok_r, wp_r, nl_r: (be_r[t], 0, h)),
            pl.BlockSpec((1, 1, _HB),
                         lambda h, t, be_r, tok_r, wp_r, nl_r: (be_r[t], 0, h)),
            pl.BlockSpec((1, 1, _DIM),
                         lambda h, t, be_r, tok_r, wp_r, nl_r: (be_r[t], 0, 0)),
        ],
        out_specs=pl.BlockSpec((_N, _DIM),
                               lambda h, t, be_r, tok_r, wp_r, nl_r: (0, 0)),
        scratch_shapes=[
            pltpu.VMEM((_BT, _DIM), jnp.float32),
            pltpu.VMEM((_BT, _DIM), jnp.float32),
            pltpu.VMEM((_BT, _DIM), jnp.float32),
        ],
    )
    out = pl.pallas_call(
        _ffn_body,
        grid_spec=grid_spec,
        out_shape=jax.ShapeDtypeStruct((_N, _DIM), jnp.float32),
    )(be, tokpad, wpad, n_live[None], x, W1, W2,
      b1.reshape(_E, 1, _HID), b2.reshape(_E, 1, _DIM))

    return out, loss
